# baseline (speedup 1.0000x reference)
import functools

import jax
import jax.numpy as jnp
from jax import lax
from jax.experimental import pallas as pl
from jax.experimental.pallas import tpu as pltpu
from jax.experimental.pallas import tpu_sc as plsc

BATCH = 16384
EMBED = 32
HIDDEN = 64
ROLES = 16
PEDS = 8

NC = 2
NS = 16
NW = NC * NS
BPW = BATCH // NW
CH = 32
NCHUNK = BPW // CH
LANES = 16
WIDE = 128
PACK = WIDE // EMBED


TILE_H = 8


NG = BPW // LANES


def _fire_group(idx_v, tab_hbm, buf_v, sem, g):
    vec = idx_v[pl.ds(g * LANES, LANES)]
    tvec = lax.shift_right_logical(vec, 3)
    for j in range(LANES):
        pltpu.async_copy(tab_hbm.at[tvec[j]],
                         buf_v.at[pl.ds(j * TILE_H, TILE_H)], sem)


def _drain_extract_group(idx_v, tab_hbm, buf_v, out_v, sem, g):
    iota = lax.iota(jnp.int32, LANES)
    for j in range(LANES):
        pltpu.make_async_copy(
            tab_hbm.at[0],
            buf_v.at[pl.ds(j * TILE_H, TILE_H)], sem).wait()
    vec = idx_v[pl.ds(g * LANES, LANES)]
    grows = g * LANES + iota
    sub = vec & 7
    rowv = iota * TILE_H + sub
    orow = lax.shift_right_logical(grows, 2)
    ocol0 = (grows & 3) * EMBED
    for w in range(EMBED):
        wv = jnp.full((LANES,), w, jnp.int32)
        val = plsc.load_gather(buf_v, [rowv, wv])
        plsc.store_scatter(out_v, [orow, ocol0 + w], val)


def _gather_body(cidx_hbm, sidx_hbm, ctab_hbm, stab_hbm,
                 cembw_hbm, sembw_hbm,
                 cidx_v, sidx_v,
                 cbuf0_v, cbuf1_v, sbuf0_v, sbuf1_v,
                 cout_v, sout_v, sem_c0, sem_c1, sem_s0, sem_s1):
    wid = lax.axis_index("s") * NC + lax.axis_index("c")
    base = pl.multiple_of(wid * BPW, BPW)
    pltpu.sync_copy(cidx_hbm.at[pl.ds(base, BPW)], cidx_v)
    pltpu.sync_copy(sidx_hbm.at[pl.ds(base, BPW)], sidx_v)

    cbufs = (cbuf0_v, cbuf1_v)
    sbufs = (sbuf0_v, sbuf1_v)
    csems = (sem_c0, sem_c1)
    ssems = (sem_s0, sem_s1)

    _fire_group(cidx_v, ctab_hbm, cbuf0_v, sem_c0, 0)
    _fire_group(sidx_v, stab_hbm, sbuf0_v, sem_s0, 0)

    def body(g2, carry):
        for b in range(2):
            grp = g2 * 2 + b
            nxt = grp + 1
            nb = 1 - b

            @pl.when(nxt < NG)
            def _():
                _fire_group(cidx_v, ctab_hbm, cbufs[nb], csems[nb], nxt)
                _fire_group(sidx_v, stab_hbm, sbufs[nb], ssems[nb], nxt)
            _drain_extract_group(cidx_v, ctab_hbm, cbufs[b], cout_v,
                                 csems[b], grp)
            _drain_extract_group(sidx_v, stab_hbm, sbufs[b], sout_v,
                                 ssems[b], grp)
        return carry
    lax.fori_loop(0, NG // 2, body, 0)

    obase = pl.multiple_of(base // PACK, BPW // PACK)
    pltpu.sync_copy(cout_v, cembw_hbm.at[pl.ds(obase, BPW // PACK)])
    pltpu.sync_copy(sout_v, sembw_hbm.at[pl.ds(obase, BPW // PACK)])


@functools.cache
def _make_gather():
    return pl.kernel(
        _gather_body,
        out_type=(
            jax.ShapeDtypeStruct((BATCH // PACK, WIDE), jnp.float32),
            jax.ShapeDtypeStruct((BATCH // PACK, WIDE), jnp.float32),
        ),
        mesh=plsc.VectorSubcoreMesh(
            core_axis_name="c", subcore_axis_name="s",
            num_cores=NC, num_subcores=NS,
        ),
        scratch_types=[
            pltpu.VMEM((BPW,), jnp.int32),
            pltpu.VMEM((BPW,), jnp.int32),
            pltpu.VMEM((LANES * TILE_H, EMBED), jnp.float32),
            pltpu.VMEM((LANES * TILE_H, EMBED), jnp.float32),
            pltpu.VMEM((LANES * TILE_H, EMBED), jnp.float32),
            pltpu.VMEM((LANES * TILE_H, EMBED), jnp.float32),
            pltpu.VMEM((BPW // PACK, WIDE), jnp.float32),
            pltpu.VMEM((BPW // PACK, WIDE), jnp.float32),
            pltpu.SemaphoreType.DMA,
            pltpu.SemaphoreType.DMA,
            pltpu.SemaphoreType.DMA,
            pltpu.SemaphoreType.DMA,
        ],
        compiler_params=pltpu.CompilerParams(needs_layout_passes=False,
                                             use_tc_tiling_on_sc=False),
    )


BLK4 = 512


def _mlp_body(c_ref, s_ref, wh_ref, bh_ref, wr_ref, br_ref, wp_ref, bp_ref,
              role_ref, ped_ref):
    cw = c_ref[...]
    sw = s_ref[...]
    wh = wh_ref[...]
    top = wh[:EMBED, :]
    bot = wh[EMBED:, :]
    for k in range(PACK):
        c = cw[:, k * EMBED:(k + 1) * EMBED]
        s = sw[:, k * EMBED:(k + 1) * EMBED]
        h = jnp.dot(c, top, preferred_element_type=jnp.float32)
        h = h + jnp.dot(s, bot, preferred_element_type=jnp.float32)
        h = jnp.maximum(h + bh_ref[...], 0.0)
        role_ref[:, k * ROLES:(k + 1) * ROLES] = (
            jnp.dot(h, wr_ref[...], preferred_element_type=jnp.float32)
            + br_ref[...])
        ped_ref[:, k * PEDS:(k + 1) * PEDS] = (
            jnp.dot(h, wp_ref[...], preferred_element_type=jnp.float32)
            + bp_ref[...])


def _mlp(cembw, sembw, W_h, b_h2, W_r, b_r2, W_p, b_p2, interpret=False):
    rep = lambda shape: pl.BlockSpec(shape, lambda i: (0, 0))
    nwide = BATCH // PACK
    return pl.pallas_call(
        _mlp_body,
        grid=(nwide // BLK4,),
        in_specs=[
            pl.BlockSpec((BLK4, WIDE), lambda i: (i, 0)),
            pl.BlockSpec((BLK4, WIDE), lambda i: (i, 0)),
            rep((2 * EMBED, HIDDEN)),
            rep((1, HIDDEN)),
            rep((HIDDEN, ROLES)),
            rep((1, ROLES)),
            rep((HIDDEN, PEDS)),
            rep((1, PEDS)),
        ],
        out_specs=[
            pl.BlockSpec((BLK4, PACK * ROLES), lambda i: (i, 0)),
            pl.BlockSpec((BLK4, PACK * PEDS), lambda i: (i, 0)),
        ],
        out_shape=[
            jax.ShapeDtypeStruct((nwide, PACK * ROLES), jnp.float32),
            jax.ShapeDtypeStruct((nwide, PACK * PEDS), jnp.float32),
        ],
        interpret=interpret,
    )(cembw, sembw, W_h, b_h2, W_r, b_r2, W_p, b_p2)


def kernel(concept_idx, style_idx, concept_table, style_table,
           W_h, b_h, W_r, b_r, W_p, b_p):
    cembw, sembw = _make_gather()(concept_idx.astype(jnp.int32),
                                  style_idx.astype(jnp.int32),
                                  concept_table.reshape(-1, TILE_H, EMBED),
                                  style_table.reshape(-1, TILE_H, EMBED))
    role_w, ped_w = _mlp(cembw, sembw, W_h, b_h.reshape(1, HIDDEN),
                         W_r, b_r.reshape(1, ROLES),
                         W_p, b_p.reshape(1, PEDS))
    return (role_w.reshape(BATCH, ROLES), ped_w.reshape(BATCH, PEDS))


# speedup vs baseline: 2.1074x; 2.1074x over previous
import functools

import jax
import jax.numpy as jnp
from jax import lax
from jax.experimental import pallas as pl
from jax.experimental.pallas import tpu as pltpu
from jax.experimental.pallas import tpu_sc as plsc

BATCH = 16384
EMBED = 32
HIDDEN = 64
ROLES = 16
PEDS = 8

NC = 2
NS = 16
NW = NC * NS
BPW = BATCH // NW
CH = 32
NCHUNK = BPW // CH
LANES = 16
WIDE = 128
PACK = WIDE // EMBED


TILE_H = 8


NG = BPW // LANES


def _fire_group(idx_v, tab_hbm, buf_v, sem, g):
    vec = idx_v[pl.ds(g * LANES, LANES)]
    tvec = lax.shift_right_logical(vec, 3)
    for j in range(LANES):
        pltpu.async_copy(tab_hbm.at[tvec[j]],
                         buf_v.at[pl.ds(j * TILE_H, TILE_H)], sem)


def _drain_extract_group(idx_v, tab_hbm, buf_v, out_v, sem, g):
    iota = lax.iota(jnp.int32, LANES)
    for j in range(LANES):
        pltpu.make_async_copy(
            tab_hbm.at[0],
            buf_v.at[pl.ds(j * TILE_H, TILE_H)], sem).wait()
    vec = idx_v[pl.ds(g * LANES, LANES)]
    grows = g * LANES + iota
    sub = vec & 7
    rowv = iota * TILE_H + sub
    orow = lax.shift_right_logical(grows, 2)
    ocol0 = (grows & 3) * EMBED
    for w in range(EMBED):
        wv = jnp.full((LANES,), w, jnp.int32)
        val = plsc.load_gather(buf_v, [rowv, wv])
        plsc.store_scatter(out_v, [orow, ocol0 + w], val)


def _gather_body(cidx_hbm, sidx_hbm, ctab_hbm, stab_hbm,
                 cembw_hbm, sembw_hbm,
                 cidx_v, sidx_v,
                 cbuf0_v, cbuf1_v, sbuf0_v, sbuf1_v,
                 cout_v, sout_v, sem_c0, sem_c1, sem_s0, sem_s1):
    wid = lax.axis_index("s") * NC + lax.axis_index("c")
    base = pl.multiple_of(wid * BPW, BPW)
    pltpu.sync_copy(cidx_hbm.at[pl.ds(base, BPW)], cidx_v)
    pltpu.sync_copy(sidx_hbm.at[pl.ds(base, BPW)], sidx_v)

    cbufs = (cbuf0_v, cbuf1_v)
    sbufs = (sbuf0_v, sbuf1_v)
    csems = (sem_c0, sem_c1)
    ssems = (sem_s0, sem_s1)

    _fire_group(cidx_v, ctab_hbm, cbuf0_v, sem_c0, 0)
    _fire_group(sidx_v, stab_hbm, sbuf0_v, sem_s0, 0)

    def body(g2, carry):
        for b in range(2):
            grp = g2 * 2 + b
            nxt = grp + 1
            nb = 1 - b

            @pl.when(nxt < NG)
            def _():
                _fire_group(cidx_v, ctab_hbm, cbufs[nb], csems[nb], nxt)
                _fire_group(sidx_v, stab_hbm, sbufs[nb], ssems[nb], nxt)
            _drain_extract_group(cidx_v, ctab_hbm, cbufs[b], cout_v,
                                 csems[b], grp)
            _drain_extract_group(sidx_v, stab_hbm, sbufs[b], sout_v,
                                 ssems[b], grp)
        return carry
    lax.fori_loop(0, NG // 2, body, 0)

    obase = pl.multiple_of(base // PACK, BPW // PACK)
    pltpu.sync_copy(cout_v, cembw_hbm.at[pl.ds(obase, BPW // PACK)])
    pltpu.sync_copy(sout_v, sembw_hbm.at[pl.ds(obase, BPW // PACK)])


@functools.cache
def _make_gather():
    return pl.kernel(
        _gather_body,
        out_type=(
            jax.ShapeDtypeStruct((BATCH // PACK, WIDE), jnp.float32),
            jax.ShapeDtypeStruct((BATCH // PACK, WIDE), jnp.float32),
        ),
        mesh=plsc.VectorSubcoreMesh(
            core_axis_name="c", subcore_axis_name="s",
            num_cores=NC, num_subcores=NS,
        ),
        scratch_types=[
            pltpu.VMEM((BPW,), jnp.int32),
            pltpu.VMEM((BPW,), jnp.int32),
            pltpu.VMEM((LANES * TILE_H, EMBED), jnp.float32),
            pltpu.VMEM((LANES * TILE_H, EMBED), jnp.float32),
            pltpu.VMEM((LANES * TILE_H, EMBED), jnp.float32),
            pltpu.VMEM((LANES * TILE_H, EMBED), jnp.float32),
            pltpu.VMEM((BPW // PACK, WIDE), jnp.float32),
            pltpu.VMEM((BPW // PACK, WIDE), jnp.float32),
            pltpu.SemaphoreType.DMA,
            pltpu.SemaphoreType.DMA,
            pltpu.SemaphoreType.DMA,
            pltpu.SemaphoreType.DMA,
        ],
        compiler_params=pltpu.CompilerParams(needs_layout_passes=False),
    )


BLK4 = 512


def _mlp_body(c_ref, s_ref, wh_ref, bh_ref, wr_ref, br_ref, wp_ref, bp_ref,
              role_ref, ped_ref):
    cw = c_ref[...]
    sw = s_ref[...]
    wh = wh_ref[...]
    top = wh[:EMBED, :]
    bot = wh[EMBED:, :]
    for k in range(PACK):
        c = cw[:, k * EMBED:(k + 1) * EMBED]
        s = sw[:, k * EMBED:(k + 1) * EMBED]
        h = jnp.dot(c, top, preferred_element_type=jnp.float32)
        h = h + jnp.dot(s, bot, preferred_element_type=jnp.float32)
        h = jnp.maximum(h + bh_ref[...], 0.0)
        role_ref[:, k * ROLES:(k + 1) * ROLES] = (
            jnp.dot(h, wr_ref[...], preferred_element_type=jnp.float32)
            + br_ref[...])
        ped_ref[:, k * PEDS:(k + 1) * PEDS] = (
            jnp.dot(h, wp_ref[...], preferred_element_type=jnp.float32)
            + bp_ref[...])


def _mlp(cembw, sembw, W_h, b_h2, W_r, b_r2, W_p, b_p2, interpret=False):
    rep = lambda shape: pl.BlockSpec(shape, lambda i: (0, 0))
    nwide = BATCH // PACK
    return pl.pallas_call(
        _mlp_body,
        grid=(nwide // BLK4,),
        in_specs=[
            pl.BlockSpec((BLK4, WIDE), lambda i: (i, 0)),
            pl.BlockSpec((BLK4, WIDE), lambda i: (i, 0)),
            rep((2 * EMBED, HIDDEN)),
            rep((1, HIDDEN)),
            rep((HIDDEN, ROLES)),
            rep((1, ROLES)),
            rep((HIDDEN, PEDS)),
            rep((1, PEDS)),
        ],
        out_specs=[
            pl.BlockSpec((BLK4, PACK * ROLES), lambda i: (i, 0)),
            pl.BlockSpec((BLK4, PACK * PEDS), lambda i: (i, 0)),
        ],
        out_shape=[
            jax.ShapeDtypeStruct((nwide, PACK * ROLES), jnp.float32),
            jax.ShapeDtypeStruct((nwide, PACK * PEDS), jnp.float32),
        ],
        interpret=interpret,
    )(cembw, sembw, W_h, b_h2, W_r, b_r2, W_p, b_p2)


def kernel(concept_idx, style_idx, concept_table, style_table,
           W_h, b_h, W_r, b_r, W_p, b_p):
    cembw, sembw = _make_gather()(concept_idx.astype(jnp.int32),
                                  style_idx.astype(jnp.int32),
                                  concept_table.reshape(-1, TILE_H, EMBED),
                                  style_table.reshape(-1, TILE_H, EMBED))
    role_w, ped_w = _mlp(cembw, sembw, W_h, b_h.reshape(1, HIDDEN),
                         W_r, b_r.reshape(1, ROLES),
                         W_p, b_p.reshape(1, PEDS))
    return (role_w.reshape(BATCH, ROLES), ped_w.reshape(BATCH, PEDS))
